# baseline (speedup 1.0000x reference)
import functools

import jax
import jax.numpy as jnp
from jax import lax
from jax.experimental import pallas as pl
from jax.experimental.pallas import tpu as pltpu
from jax.experimental.pallas import tpu_sc as plsc

N, E, C, D, O = 10000, 320000, 128, 16, 64
C2 = 2 * C

NW = 32
EW = E // NW
CHUNK = 80
NCH = EW // CHUNK
S = 5
ES = E // S
EWS = ES // NW
NCHS = EWS // CHUNK
NPAD = 10240
ROWS_PER_SUB = NPAD // 16
ZROWS = 128

@functools.cache
def _sc_mesh():
    return plsc.VectorSubcoreMesh(core_axis_name="c", subcore_axis_name="s",
                                  num_cores=2, num_subcores=16)


BN = 1000


def _pack_gates(t):
    lo = jax.lax.bitcast_convert_type(
        t[:, :C].astype(jnp.bfloat16), jnp.uint16).astype(jnp.uint32)
    hi = jax.lax.bitcast_convert_type(
        t[:, C:].astype(jnp.bfloat16), jnp.uint16).astype(jnp.uint32)
    return jax.lax.bitcast_convert_type(lo | (hi << 16), jnp.int32)


def _unpack_gates(u):
    ub = jax.lax.bitcast_convert_type(u, jnp.uint32)
    f = jax.lax.bitcast_convert_type(ub << 16, jnp.float32)
    s = jax.lax.bitcast_convert_type(ub & jnp.uint32(0xFFFF0000), jnp.float32)
    return f, s


def _tables1_body(x_ref, wp_ref, bp_ref, wq_ref, p_ref, q_ref):
    x = x_ref[...]
    p = jnp.dot(x, wp_ref[...], preferred_element_type=jnp.float32) + bp_ref[...]
    q = jnp.dot(x, wq_ref[...], preferred_element_type=jnp.float32)
    p_ref[...] = _pack_gates(p)
    q_ref[...] = _pack_gates(q)


def _tc_tables1(x, wp, bp, wq):
    grid = N // BN
    return pl.pallas_call(
        _tables1_body,
        grid=(grid,),
        in_specs=[
            pl.BlockSpec((BN, C), lambda i: (i, 0)),
            pl.BlockSpec((C, C2), lambda i: (0, 0)),
            pl.BlockSpec((1, C2), lambda i: (0, 0)),
            pl.BlockSpec((C, C2), lambda i: (0, 0)),
        ],
        out_specs=[
            pl.BlockSpec((BN, C), lambda i: (i, 0)),
            pl.BlockSpec((BN, C), lambda i: (i, 0)),
        ],
        out_shape=[
            jax.ShapeDtypeStruct((N, C), jnp.int32),
            jax.ShapeDtypeStruct((N, C), jnp.int32),
        ],
    )(x, wp, bp, wq)


def _tables2_body(x_ref, a0_ref, a1_ref, a2_ref, a3_ref, wp_ref, bp_ref,
                  wq_ref, p_ref, q_ref, hsum_ref):
    i = pl.program_id(0)
    h = jnp.maximum(
        x_ref[...] + (a0_ref[...] + a1_ref[...]) + (a2_ref[...] + a3_ref[...]),
        0.0)
    p = jnp.dot(h, wp_ref[...], preferred_element_type=jnp.float32) + bp_ref[...]
    q = jnp.dot(h, wq_ref[...], preferred_element_type=jnp.float32)
    p_ref[...] = _pack_gates(p)
    q_ref[...] = _pack_gates(q)
    part = jnp.sum(h, axis=0, keepdims=True)

    @pl.when(i == 0)
    def _():
        hsum_ref[...] = part

    @pl.when(i != 0)
    def _():
        hsum_ref[...] += part


def _tc_tables2(x, a0, a1, a2, a3, wp, bp, wq):
    grid = N // BN
    return pl.pallas_call(
        _tables2_body,
        grid=(grid,),
        in_specs=[
            pl.BlockSpec((BN, C), lambda i: (i, 0)),
            pl.BlockSpec((BN, C), lambda i: (i, 0)),
            pl.BlockSpec((BN, C), lambda i: (i, 0)),
            pl.BlockSpec((BN, C), lambda i: (i, 0)),
            pl.BlockSpec((BN, C), lambda i: (i, 0)),
            pl.BlockSpec((C, C2), lambda i: (0, 0)),
            pl.BlockSpec((1, C2), lambda i: (0, 0)),
            pl.BlockSpec((C, C2), lambda i: (0, 0)),
        ],
        out_specs=[
            pl.BlockSpec((BN, C), lambda i: (i, 0)),
            pl.BlockSpec((BN, C), lambda i: (i, 0)),
            pl.BlockSpec((1, C), lambda i: (0, 0)),
        ],
        out_shape=[
            jax.ShapeDtypeStruct((N, C), jnp.int32),
            jax.ShapeDtypeStruct((N, C), jnp.int32),
            jax.ShapeDtypeStruct((1, C), jnp.float32),
        ],
    )(x, a0, a1, a2, a3, wp, bp, wq)


BE = 1000


def _softplus(s):
    return jnp.maximum(s, 0.0) + jnp.log1p(jnp.exp(-jnp.abs(s)))


def _sigmoid(f):
    return 0.5 * jnp.tanh(f * 0.5) + 0.5


def _edge_m_body(g_ref, ea_ref, we_ref, m_ref):
    gf, gs = _unpack_gates(g_ref[...])
    t = jnp.dot(ea_ref[...], we_ref[...], preferred_element_type=jnp.float32)
    f = gf + t[:, :C]
    s = gs + t[:, C:]
    m_ref[...] = _sigmoid(f) * _softplus(s)


def _tc_edge_m(g, ea, we, s):
    ne = g.shape[0]
    grid = ne // BE
    soff = s * (ES // BE)
    return pl.pallas_call(
        _edge_m_body,
        grid=(grid,),
        in_specs=[
            pl.BlockSpec((BE, C), lambda i: (i, 0)),
            pl.BlockSpec((BE, D), lambda i: (soff + i, 0)),
            pl.BlockSpec((D, C2), lambda i: (0, 0)),
        ],
        out_specs=pl.BlockSpec((BE, C), lambda i: (i, 0)),
        out_shape=jax.ShapeDtypeStruct((ne, C), jnp.float32),
    )(g, ea, we)


def _edge_msum_body(g_ref, ea_ref, we_ref, msum_ref):
    i = pl.program_id(0)
    gf, gs = _unpack_gates(g_ref[...])
    t = jnp.dot(ea_ref[...], we_ref[...], preferred_element_type=jnp.float32)
    f = gf + t[:, :C]
    s = gs + t[:, C:]
    part = jnp.sum(_sigmoid(f) * _softplus(s), axis=0, keepdims=True)

    @pl.when(i == 0)
    def _():
        msum_ref[...] = part

    @pl.when(i != 0)
    def _():
        msum_ref[...] += part


def _tc_edge_msum(g, ea, we, s):
    grid = g.shape[0] // BE
    soff = s * (ES // BE)
    return pl.pallas_call(
        _edge_msum_body,
        grid=(grid,),
        in_specs=[
            pl.BlockSpec((BE, C), lambda i: (i, 0)),
            pl.BlockSpec((BE, D), lambda i: (soff + i, 0)),
            pl.BlockSpec((D, C2), lambda i: (0, 0)),
        ],
        out_specs=pl.BlockSpec((1, C), lambda i: (0, 0)),
        out_shape=jax.ShapeDtypeStruct((1, C), jnp.float32),
    )(g, ea, we)


def _readout_body(sums_ref, wn_ref, nb_ref, o_ref):
    s = jnp.sum(sums_ref[...], axis=0, keepdims=True)
    o_ref[...] = jnp.dot(s, wn_ref[...], preferred_element_type=jnp.float32) + nb_ref[...]


def _tc_readout(sums, wnt, nb):
    nrows = sums.shape[0]
    return pl.pallas_call(
        _readout_body,
        in_specs=[
            pl.BlockSpec((nrows, C), lambda: (0, 0)),
            pl.BlockSpec((C, O), lambda: (0, 0)),
            pl.BlockSpec((1, O), lambda: (0, 0)),
        ],
        out_specs=pl.BlockSpec((1, O), lambda: (0, 0)),
        out_shape=jax.ShapeDtypeStruct((1, O), jnp.float32),
    )(sums, wnt, nb)


def _g_add_rows(abuf, bbuf):
    hi_mask = jnp.int32(-65536)
    bc = lax.bitcast_convert_type

    def add_row(e, _):
        for j in range(C // 16):
            a = abuf[e, pl.ds(j * 16, 16)]
            b = bbuf[e, pl.ds(j * 16, 16)]
            af = bc(a << 16, jnp.float32)
            as_ = bc(a & hi_mask, jnp.float32)
            bf_ = bc(b << 16, jnp.float32)
            bs_ = bc(b & hi_mask, jnp.float32)
            fu = bc(af + bf_, jnp.int32)
            su = bc(as_ + bs_, jnp.int32)
            abuf[e, pl.ds(j * 16, 16)] = (
                lax.shift_right_logical(fu, 16) | (su & hi_mask))
        return 0

    lax.fori_loop(0, CHUNK, add_row, 0)


def _make_sc_gather_body(slice_idx):
    def body(dst_hbm, src_hbm, p_hbm, q_hbm, g_hbm,
             didx, sidx, a0, a1, b0, b1,
             sem_a0, sem_a1, sem_b0, sem_b1, sem_s0, sem_s1):
        return _sc_gather_impl(
            slice_idx, dst_hbm, src_hbm, p_hbm, q_hbm, g_hbm,
            didx, sidx, a0, a1, b0, b1,
            sem_a0, sem_a1, sem_b0, sem_b1, sem_s0, sem_s1)
    return body


def _sc_gather_impl(slice_idx, dst_hbm, src_hbm, p_hbm, q_hbm, g_hbm,
                    didx, sidx, a0, a1, b0, b1,
                    sem_a0, sem_a1, sem_b0, sem_b1, sem_s0, sem_s1):
    wid = lax.axis_index("s") * 2 + lax.axis_index("c")
    pltpu.sync_copy(dst_hbm.at[slice_idx, wid], didx)
    pltpu.sync_copy(src_hbm.at[slice_idx, wid], sidx)

    bufs = ((a0, b0, sem_a0, sem_b0, sem_s0),
            (a1, b1, sem_a1, sem_b1, sem_s1))

    def issue_gather(t, p):
        a, b, sa, sb, _ = bufs[p]
        pltpu.async_copy(p_hbm.at[didx.at[t]], a, sa)
        pltpu.async_copy(q_hbm.at[sidx.at[t]], b, sb)

    def wait_gather(t, p):
        a, b, sa, sb, _ = bufs[p]
        pltpu.make_async_copy(p_hbm.at[didx.at[t]], a, sa).wait()
        pltpu.make_async_copy(q_hbm.at[sidx.at[t]], b, sb).wait()

    def wait_store(p):
        a, _, _, _, ss = bufs[p]
        pltpu.make_async_copy(a, g_hbm.at[pl.ds(0, CHUNK)], ss).wait()

    issue_gather(0, 0)

    def step(t, p, issue_next=True):
        q = 1 - p
        if issue_next:
            @pl.when(t >= 1)
            def _():
                wait_store(q)
            issue_gather(t + 1, q)

        wait_gather(t, p)
        a = bufs[p][0]
        _g_add_rows(a, bufs[p][1])
        pltpu.async_copy(a, g_hbm.at[pl.ds(wid * EWS + t * CHUNK, CHUNK)],
                         bufs[p][4])

    def pair(u, _):
        step(2 * u, 0)
        step(2 * u + 1, 1)
        return 0

    lax.fori_loop(0, NCHS // 2, pair, 0)
    step(NCHS - 1, 0, issue_next=False)
    wait_store(1)
    wait_store(0)


@functools.cache
def _sc_gather_kernel(slice_idx):
    return pl.kernel(
        _make_sc_gather_body(slice_idx),
        out_type=jax.ShapeDtypeStruct((ES, C), jnp.int32),
        mesh=_sc_mesh(),
        scratch_types=[
            pltpu.VMEM((NCHS, CHUNK), jnp.int32),
            pltpu.VMEM((NCHS, CHUNK), jnp.int32),
            pltpu.VMEM((CHUNK, C), jnp.int32),
            pltpu.VMEM((CHUNK, C), jnp.int32),
            pltpu.VMEM((CHUNK, C), jnp.int32),
            pltpu.VMEM((CHUNK, C), jnp.int32),
            pltpu.SemaphoreType.DMA,
            pltpu.SemaphoreType.DMA,
            pltpu.SemaphoreType.DMA,
            pltpu.SemaphoreType.DMA,
            pltpu.SemaphoreType.DMA,
            pltpu.SemaphoreType.DMA,
        ],
    )


def _sc_gather(s, dst4, src4, p, q):
    return _sc_gather_kernel(s)(dst4, src4, p, q)


def _make_sc_scatter_body(slices):
    k = len(slices)

    def body(dst_hbm, *rest):
        ms = rest[:k]
        out_hbm = rest[k]
        didx, zbuf, mb0, mb1, agg, sem_m0, sem_m1 = rest[k + 1:]
        cid = lax.axis_index("c")
        sid = lax.axis_index("s")
        wid = sid * 2 + cid

        def zrow(r, _):
            for j in range(C // 16):
                zbuf[r, pl.ds(j * 16, 16)] = jnp.zeros((16,), jnp.float32)
            return 0

        lax.fori_loop(0, ZROWS, zrow, 0)
        for rep in range(ROWS_PER_SUB // ZROWS):
            pltpu.sync_copy(
                zbuf, agg.at[pl.ds(sid * ROWS_PER_SUB + rep * ZROWS, ZROWS)])
        plsc.subcore_barrier()

        bufs = ((mb0, sem_m0), (mb1, sem_m1))

        for s, m_hbm in zip(slices, ms):
            pltpu.sync_copy(dst_hbm.at[s, wid], didx)
            def issue_load(t, p, m_hbm=m_hbm):
                b, sm = bufs[p]
                pltpu.async_copy(
                    m_hbm.at[pl.ds(wid * EWS + t * CHUNK, CHUNK)], b, sm)

            def wait_load(p, m_hbm=m_hbm):
                b, sm = bufs[p]
                pltpu.make_async_copy(m_hbm.at[pl.ds(0, CHUNK)], b, sm).wait()

            def step(t, p, issue_load=issue_load, wait_load=wait_load,
                     issue_next=True):
                if issue_next:
                    issue_load(t + 1, 1 - p)
                wait_load(p)
                pltpu.sync_copy(bufs[p][0], agg.at[didx.at[t]], add=True)

            issue_load(0, 0)

            def pair(u, _, step=step):
                step(2 * u, 0)
                step(2 * u + 1, 1)
                return 0

            lax.fori_loop(0, NCHS // 2, pair, 0)
            step(NCHS - 1, 0, issue_next=False)
        plsc.subcore_barrier()

        pltpu.sync_copy(
            agg.at[pl.ds(sid * ROWS_PER_SUB, ROWS_PER_SUB)],
            out_hbm.at[cid, pl.ds(sid * ROWS_PER_SUB, ROWS_PER_SUB)])

    return body


@functools.cache
def _sc_scatter_kernel(slices):
    return pl.kernel(
        _make_sc_scatter_body(slices),
        out_type=jax.ShapeDtypeStruct((2, NPAD, C), jnp.float32),
        mesh=_sc_mesh(),
        scratch_types=[
            pltpu.VMEM((NCHS, CHUNK), jnp.int32),
            pltpu.VMEM((ZROWS, C), jnp.float32),
            pltpu.VMEM((CHUNK, C), jnp.float32),
            pltpu.VMEM((CHUNK, C), jnp.float32),
            pltpu.VMEM_SHARED((NPAD, C), jnp.float32),
            pltpu.SemaphoreType.DMA,
            pltpu.SemaphoreType.DMA,
        ],
    )


def _sc_scatter(slices, dst4, ms):
    return _sc_scatter_kernel(tuple(slices))(dst4, *ms)


def kernel(x, edge_index, edge_attr, Wf1, bf1, Ws1, bs1, Wf2, bf2, Ws2, bs2, Wn, node_bias):
    dst = edge_index[1].astype(jnp.int32)
    src = edge_index[0].astype(jnp.int32)
    dst4 = dst.reshape(S, NW, NCHS, CHUNK)
    src4 = src.reshape(S, NW, NCHS, CHUNK)

    def split_w(Wf, Ws):
        wp = jnp.concatenate([Wf[:, :C].T, Ws[:, :C].T], axis=1)
        wq = jnp.concatenate([Wf[:, C:C2].T, Ws[:, C:C2].T], axis=1)
        we = jnp.concatenate([Wf[:, C2:].T, Ws[:, C2:].T], axis=1)
        return wp, wq, we

    wp1, wq1, we1 = split_w(Wf1, Ws1)
    wp2, wq2, we2 = split_w(Wf2, Ws2)
    bp1 = jnp.concatenate([bf1, bs1]).reshape(1, C2)
    bp2 = jnp.concatenate([bf2, bs2]).reshape(1, C2)

    p1, q1 = _tc_tables1(x, wp1, bp1, wq1)
    ms = []
    for s in range(S):
        g1 = _sc_gather(s, dst4, src4, p1, q1)
        ms.append(_tc_edge_m(g1, edge_attr, we1, s))
    parts_a = _sc_scatter(range(3), dst4, ms[:3])
    parts_b = _sc_scatter(range(3, S), dst4, ms[3:])

    p2, q2, hsum = _tc_tables2(x, parts_a[0], parts_a[1],
                               parts_b[0], parts_b[1], wp2, bp2, wq2)
    sums = [hsum]
    for s in range(S):
        g2 = _sc_gather(s, dst4, src4, p2, q2)
        sums.append(_tc_edge_msum(g2, edge_attr, we2, s))

    return _tc_readout(jnp.concatenate(sums, axis=0), Wn.T,
                       node_bias.reshape(1, O))


# speedup vs baseline: 1.1720x; 1.1720x over previous
import functools

import jax
import jax.numpy as jnp
from jax import lax
from jax.experimental import pallas as pl
from jax.experimental.pallas import tpu as pltpu
from jax.experimental.pallas import tpu_sc as plsc

N, E, C, D, O = 10000, 320000, 128, 16, 64
C2 = 2 * C

NW = 32
EW = E // NW
CHUNK = 80
NCH = EW // CHUNK
S = 5
ES = E // S
EWS = ES // NW
NCHS = EWS // CHUNK
NPAD = 10240
ROWS_PER_SUB = NPAD // 16
ZROWS = 128

@functools.cache
def _sc_mesh():
    return plsc.VectorSubcoreMesh(core_axis_name="c", subcore_axis_name="s",
                                  num_cores=2, num_subcores=16)


BN = 1000


def _pack_gates(t):
    lo = jax.lax.bitcast_convert_type(
        t[:, :C].astype(jnp.bfloat16), jnp.uint16).astype(jnp.uint32)
    hi = jax.lax.bitcast_convert_type(
        t[:, C:].astype(jnp.bfloat16), jnp.uint16).astype(jnp.uint32)
    return jax.lax.bitcast_convert_type(lo | (hi << 16), jnp.int32)


def _unpack_gates(u):
    ub = jax.lax.bitcast_convert_type(u, jnp.uint32)
    f = jax.lax.bitcast_convert_type(ub << 16, jnp.float32)
    s = jax.lax.bitcast_convert_type(ub & jnp.uint32(0xFFFF0000), jnp.float32)
    return f, s


def _tables1_body(x_ref, wp_ref, bp_ref, wq_ref, p_ref, q_ref):
    x = x_ref[...]
    p = jnp.dot(x, wp_ref[...], preferred_element_type=jnp.float32) + bp_ref[...]
    q = jnp.dot(x, wq_ref[...], preferred_element_type=jnp.float32)
    p_ref[...] = _pack_gates(p)
    q_ref[...] = _pack_gates(q)


def _tc_tables1(x, wp, bp, wq):
    grid = N // BN
    return pl.pallas_call(
        _tables1_body,
        grid=(grid,),
        in_specs=[
            pl.BlockSpec((BN, C), lambda i: (i, 0)),
            pl.BlockSpec((C, C2), lambda i: (0, 0)),
            pl.BlockSpec((1, C2), lambda i: (0, 0)),
            pl.BlockSpec((C, C2), lambda i: (0, 0)),
        ],
        out_specs=[
            pl.BlockSpec((BN, C), lambda i: (i, 0)),
            pl.BlockSpec((BN, C), lambda i: (i, 0)),
        ],
        out_shape=[
            jax.ShapeDtypeStruct((N, C), jnp.int32),
            jax.ShapeDtypeStruct((N, C), jnp.int32),
        ],
    )(x, wp, bp, wq)


def _tables2_body(x_ref, a0_ref, a1_ref, a2_ref, a3_ref, wp_ref, bp_ref,
                  wq_ref, p_ref, q_ref, hsum_ref):
    i = pl.program_id(0)
    h = jnp.maximum(
        x_ref[...] + (a0_ref[...] + a1_ref[...]) + (a2_ref[...] + a3_ref[...]),
        0.0)
    p = jnp.dot(h, wp_ref[...], preferred_element_type=jnp.float32) + bp_ref[...]
    q = jnp.dot(h, wq_ref[...], preferred_element_type=jnp.float32)
    p_ref[...] = _pack_gates(p)
    q_ref[...] = _pack_gates(q)
    part = jnp.sum(h, axis=0, keepdims=True)

    @pl.when(i == 0)
    def _():
        hsum_ref[...] = part

    @pl.when(i != 0)
    def _():
        hsum_ref[...] += part


def _tc_tables2(x, a0, a1, a2, a3, wp, bp, wq):
    grid = N // BN
    return pl.pallas_call(
        _tables2_body,
        grid=(grid,),
        in_specs=[
            pl.BlockSpec((BN, C), lambda i: (i, 0)),
            pl.BlockSpec((BN, C), lambda i: (i, 0)),
            pl.BlockSpec((BN, C), lambda i: (i, 0)),
            pl.BlockSpec((BN, C), lambda i: (i, 0)),
            pl.BlockSpec((BN, C), lambda i: (i, 0)),
            pl.BlockSpec((C, C2), lambda i: (0, 0)),
            pl.BlockSpec((1, C2), lambda i: (0, 0)),
            pl.BlockSpec((C, C2), lambda i: (0, 0)),
        ],
        out_specs=[
            pl.BlockSpec((BN, C), lambda i: (i, 0)),
            pl.BlockSpec((BN, C), lambda i: (i, 0)),
            pl.BlockSpec((1, C), lambda i: (0, 0)),
        ],
        out_shape=[
            jax.ShapeDtypeStruct((N, C), jnp.int32),
            jax.ShapeDtypeStruct((N, C), jnp.int32),
            jax.ShapeDtypeStruct((1, C), jnp.float32),
        ],
    )(x, a0, a1, a2, a3, wp, bp, wq)


BE = 2000


def _softplus(s):
    return jnp.maximum(s, 0.0) + jnp.log1p(jnp.exp(-jnp.abs(s)))


def _sigmoid(f):
    return 0.5 * jnp.tanh(f * 0.5) + 0.5


def _edge_m_body(g_ref, ea_ref, we_ref, m_ref):
    gf, gs = _unpack_gates(g_ref[...])
    t = jnp.dot(ea_ref[...], we_ref[...], preferred_element_type=jnp.float32)
    f = gf + t[:, :C]
    s = gs + t[:, C:]
    m_ref[...] = _sigmoid(f) * _softplus(s)


def _tc_edge_m(g, ea, we, s):
    ne = g.shape[0]
    grid = ne // BE
    soff = s * (ES // BE)
    return pl.pallas_call(
        _edge_m_body,
        grid=(grid,),
        in_specs=[
            pl.BlockSpec((BE, C), lambda i: (i, 0)),
            pl.BlockSpec((BE, D), lambda i: (soff + i, 0)),
            pl.BlockSpec((D, C2), lambda i: (0, 0)),
        ],
        out_specs=pl.BlockSpec((BE, C), lambda i: (i, 0)),
        out_shape=jax.ShapeDtypeStruct((ne, C), jnp.float32),
    )(g, ea, we)


def _edge_msum_body(g_ref, ea_ref, we_ref, msum_ref):
    i = pl.program_id(0)
    gf, gs = _unpack_gates(g_ref[...])
    t = jnp.dot(ea_ref[...], we_ref[...], preferred_element_type=jnp.float32)
    f = gf + t[:, :C]
    s = gs + t[:, C:]
    part = jnp.sum(_sigmoid(f) * _softplus(s), axis=0, keepdims=True)

    @pl.when(i == 0)
    def _():
        msum_ref[...] = part

    @pl.when(i != 0)
    def _():
        msum_ref[...] += part


def _tc_edge_msum(g, ea, we, s):
    grid = g.shape[0] // BE
    soff = s * (ES // BE)
    return pl.pallas_call(
        _edge_msum_body,
        grid=(grid,),
        in_specs=[
            pl.BlockSpec((BE, C), lambda i: (i, 0)),
            pl.BlockSpec((BE, D), lambda i: (soff + i, 0)),
            pl.BlockSpec((D, C2), lambda i: (0, 0)),
        ],
        out_specs=pl.BlockSpec((1, C), lambda i: (0, 0)),
        out_shape=jax.ShapeDtypeStruct((1, C), jnp.float32),
    )(g, ea, we)


def _readout_body(sums_ref, wn_ref, nb_ref, o_ref):
    s = jnp.sum(sums_ref[...], axis=0, keepdims=True)
    o_ref[...] = jnp.dot(s, wn_ref[...], preferred_element_type=jnp.float32) + nb_ref[...]


def _tc_readout(sums, wnt, nb):
    nrows = sums.shape[0]
    return pl.pallas_call(
        _readout_body,
        in_specs=[
            pl.BlockSpec((nrows, C), lambda: (0, 0)),
            pl.BlockSpec((C, O), lambda: (0, 0)),
            pl.BlockSpec((1, O), lambda: (0, 0)),
        ],
        out_specs=pl.BlockSpec((1, O), lambda: (0, 0)),
        out_shape=jax.ShapeDtypeStruct((1, O), jnp.float32),
    )(sums, wnt, nb)


def _g_add_rows(abuf, bbuf):
    hi_mask = jnp.int32(-65536)
    bc = lax.bitcast_convert_type

    def add_row(e, _):
        for j in range(C // 16):
            a = abuf[e, pl.ds(j * 16, 16)]
            b = bbuf[e, pl.ds(j * 16, 16)]
            af = bc(a << 16, jnp.float32)
            as_ = bc(a & hi_mask, jnp.float32)
            bf_ = bc(b << 16, jnp.float32)
            bs_ = bc(b & hi_mask, jnp.float32)
            fu = bc(af + bf_, jnp.int32)
            su = bc(as_ + bs_, jnp.int32)
            abuf[e, pl.ds(j * 16, 16)] = (
                lax.shift_right_logical(fu, 16) | (su & hi_mask))
        return 0

    lax.fori_loop(0, CHUNK, add_row, 0)


def _make_sc_gather_body(slice_idx):
    def body(dst_hbm, src_hbm, p_hbm, q_hbm, g_hbm,
             didx, sidx, a0, a1, b0, b1,
             sem_a0, sem_a1, sem_b0, sem_b1, sem_s0, sem_s1):
        return _sc_gather_impl(
            slice_idx, dst_hbm, src_hbm, p_hbm, q_hbm, g_hbm,
            didx, sidx, a0, a1, b0, b1,
            sem_a0, sem_a1, sem_b0, sem_b1, sem_s0, sem_s1)
    return body


def _sc_gather_impl(slice_idx, dst_hbm, src_hbm, p_hbm, q_hbm, g_hbm,
                    didx, sidx, a0, a1, b0, b1,
                    sem_a0, sem_a1, sem_b0, sem_b1, sem_s0, sem_s1):
    wid = lax.axis_index("s") * 2 + lax.axis_index("c")
    pltpu.sync_copy(dst_hbm.at[slice_idx, wid], didx)
    pltpu.sync_copy(src_hbm.at[slice_idx, wid], sidx)

    bufs = ((a0, b0, sem_a0, sem_b0, sem_s0),
            (a1, b1, sem_a1, sem_b1, sem_s1))

    def issue_gather(t, p):
        a, b, sa, sb, _ = bufs[p]
        pltpu.async_copy(p_hbm.at[didx.at[t]], a, sa)
        pltpu.async_copy(q_hbm.at[sidx.at[t]], b, sb)

    def wait_gather(t, p):
        a, b, sa, sb, _ = bufs[p]
        pltpu.make_async_copy(p_hbm.at[didx.at[t]], a, sa).wait()
        pltpu.make_async_copy(q_hbm.at[sidx.at[t]], b, sb).wait()

    def wait_store(p):
        a, _, _, _, ss = bufs[p]
        pltpu.make_async_copy(a, g_hbm.at[pl.ds(0, CHUNK)], ss).wait()

    issue_gather(0, 0)

    def step(t, p, issue_next=True):
        q = 1 - p
        if issue_next:
            @pl.when(t >= 1)
            def _():
                wait_store(q)
            issue_gather(t + 1, q)

        wait_gather(t, p)
        a = bufs[p][0]
        _g_add_rows(a, bufs[p][1])
        pltpu.async_copy(a, g_hbm.at[pl.ds(wid * EWS + t * CHUNK, CHUNK)],
                         bufs[p][4])

    def pair(u, _):
        step(2 * u, 0)
        step(2 * u + 1, 1)
        return 0

    lax.fori_loop(0, NCHS // 2, pair, 0)
    step(NCHS - 1, 0, issue_next=False)
    wait_store(1)
    wait_store(0)


@functools.cache
def _sc_gather_kernel(slice_idx):
    return pl.kernel(
        _make_sc_gather_body(slice_idx),
        out_type=jax.ShapeDtypeStruct((ES, C), jnp.int32),
        mesh=_sc_mesh(),
        scratch_types=[
            pltpu.VMEM((NCHS, CHUNK), jnp.int32),
            pltpu.VMEM((NCHS, CHUNK), jnp.int32),
            pltpu.VMEM((CHUNK, C), jnp.int32),
            pltpu.VMEM((CHUNK, C), jnp.int32),
            pltpu.VMEM((CHUNK, C), jnp.int32),
            pltpu.VMEM((CHUNK, C), jnp.int32),
            pltpu.SemaphoreType.DMA,
            pltpu.SemaphoreType.DMA,
            pltpu.SemaphoreType.DMA,
            pltpu.SemaphoreType.DMA,
            pltpu.SemaphoreType.DMA,
            pltpu.SemaphoreType.DMA,
        ],
    )


def _sc_gather(s, dst4, src4, p, q):
    return _sc_gather_kernel(s)(dst4, src4, p, q)


def _make_sc_scatter_body(slices):
    k = len(slices)

    def body(dst_hbm, *rest):
        ms = rest[:k]
        out_hbm = rest[k]
        didx, zbuf, mb0, mb1, agg, sem_m0, sem_m1 = rest[k + 1:]
        cid = lax.axis_index("c")
        sid = lax.axis_index("s")
        wid = sid * 2 + cid

        def zrow(r, _):
            for j in range(C // 16):
                zbuf[r, pl.ds(j * 16, 16)] = jnp.zeros((16,), jnp.float32)
            return 0

        lax.fori_loop(0, ZROWS, zrow, 0)
        for rep in range(ROWS_PER_SUB // ZROWS):
            pltpu.sync_copy(
                zbuf, agg.at[pl.ds(sid * ROWS_PER_SUB + rep * ZROWS, ZROWS)])
        plsc.subcore_barrier()

        bufs = ((mb0, sem_m0), (mb1, sem_m1))

        for s, m_hbm in zip(slices, ms):
            pltpu.sync_copy(dst_hbm.at[s, wid], didx)
            def issue_load(t, p, m_hbm=m_hbm):
                b, sm = bufs[p]
                pltpu.async_copy(
                    m_hbm.at[pl.ds(wid * EWS + t * CHUNK, CHUNK)], b, sm)

            def wait_load(p, m_hbm=m_hbm):
                b, sm = bufs[p]
                pltpu.make_async_copy(m_hbm.at[pl.ds(0, CHUNK)], b, sm).wait()

            def step(t, p, issue_load=issue_load, wait_load=wait_load,
                     issue_next=True):
                if issue_next:
                    issue_load(t + 1, 1 - p)
                wait_load(p)
                pltpu.sync_copy(bufs[p][0], agg.at[didx.at[t]], add=True)

            issue_load(0, 0)

            def pair(u, _, step=step):
                step(2 * u, 0)
                step(2 * u + 1, 1)
                return 0

            lax.fori_loop(0, NCHS // 2, pair, 0)
            step(NCHS - 1, 0, issue_next=False)
        plsc.subcore_barrier()

        pltpu.sync_copy(
            agg.at[pl.ds(sid * ROWS_PER_SUB, ROWS_PER_SUB)],
            out_hbm.at[cid, pl.ds(sid * ROWS_PER_SUB, ROWS_PER_SUB)])

    return body


@functools.cache
def _sc_scatter_kernel(slices):
    return pl.kernel(
        _make_sc_scatter_body(slices),
        out_type=jax.ShapeDtypeStruct((2, NPAD, C), jnp.float32),
        mesh=_sc_mesh(),
        scratch_types=[
            pltpu.VMEM((NCHS, CHUNK), jnp.int32),
            pltpu.VMEM((ZROWS, C), jnp.float32),
            pltpu.VMEM((CHUNK, C), jnp.float32),
            pltpu.VMEM((CHUNK, C), jnp.float32),
            pltpu.VMEM_SHARED((NPAD, C), jnp.float32),
            pltpu.SemaphoreType.DMA,
            pltpu.SemaphoreType.DMA,
        ],
    )


def _sc_scatter(slices, dst4, ms):
    return _sc_scatter_kernel(tuple(slices))(dst4, *ms)


def kernel(x, edge_index, edge_attr, Wf1, bf1, Ws1, bs1, Wf2, bf2, Ws2, bs2, Wn, node_bias):
    dst = edge_index[1].astype(jnp.int32)
    src = edge_index[0].astype(jnp.int32)
    dst4 = dst.reshape(S, NW, NCHS, CHUNK)
    src4 = src.reshape(S, NW, NCHS, CHUNK)

    def split_w(Wf, Ws):
        wp = jnp.concatenate([Wf[:, :C].T, Ws[:, :C].T], axis=1)
        wq = jnp.concatenate([Wf[:, C:C2].T, Ws[:, C:C2].T], axis=1)
        we = jnp.concatenate([Wf[:, C2:].T, Ws[:, C2:].T], axis=1)
        return wp, wq, we

    wp1, wq1, we1 = split_w(Wf1, Ws1)
    wp2, wq2, we2 = split_w(Wf2, Ws2)
    bp1 = jnp.concatenate([bf1, bs1]).reshape(1, C2)
    bp2 = jnp.concatenate([bf2, bs2]).reshape(1, C2)

    p1, q1 = _tc_tables1(x, wp1, bp1, wq1)
    ms = []
    for s in range(S):
        g1 = _sc_gather(s, dst4, src4, p1, q1)
        ms.append(_tc_edge_m(g1, edge_attr, we1, s))
    parts_a = _sc_scatter(range(3), dst4, ms[:3])
    parts_b = _sc_scatter(range(3, S), dst4, ms[3:])

    p2, q2, hsum = _tc_tables2(x, parts_a[0], parts_a[1],
                               parts_b[0], parts_b[1], wp2, bp2, wq2)
    sums = [hsum]
    for s in range(S):
        g2 = _sc_gather(s, dst4, src4, p2, q2)
        sums.append(_tc_edge_msum(g2, edge_attr, we2, s))

    return _tc_readout(jnp.concatenate(sums, axis=0), Wn.T,
                       node_bias.reshape(1, O))


# speedup vs baseline: 1.1952x; 1.0198x over previous
import functools

import jax
import jax.numpy as jnp
from jax import lax
from jax.experimental import pallas as pl
from jax.experimental.pallas import tpu as pltpu
from jax.experimental.pallas import tpu_sc as plsc

N, E, C, D, O = 10000, 320000, 128, 16, 64
C2 = 2 * C

NW = 32
EW = E // NW
CHUNK = 80
NCH = EW // CHUNK
S = 5
ES = E // S
EWS = ES // NW
NCHS = EWS // CHUNK
NPAD = 10240
ROWS_PER_SUB = NPAD // 16
ZROWS = 128

@functools.cache
def _sc_mesh():
    return plsc.VectorSubcoreMesh(core_axis_name="c", subcore_axis_name="s",
                                  num_cores=2, num_subcores=16)


BN = 1000


def _pack_gates(t):
    lo = jax.lax.bitcast_convert_type(
        t[:, :C].astype(jnp.bfloat16), jnp.uint16).astype(jnp.uint32)
    hi = jax.lax.bitcast_convert_type(
        t[:, C:].astype(jnp.bfloat16), jnp.uint16).astype(jnp.uint32)
    return jax.lax.bitcast_convert_type(lo | (hi << 16), jnp.int32)


def _unpack_gates(u):
    ub = jax.lax.bitcast_convert_type(u, jnp.uint32)
    f = jax.lax.bitcast_convert_type(ub << 16, jnp.float32)
    s = jax.lax.bitcast_convert_type(ub & jnp.uint32(0xFFFF0000), jnp.float32)
    return f, s


def _tables1_body(x_ref, wp_ref, bp_ref, wq_ref, p_ref, q_ref):
    x = x_ref[...]
    p = jnp.dot(x, wp_ref[...], preferred_element_type=jnp.float32) + bp_ref[...]
    q = jnp.dot(x, wq_ref[...], preferred_element_type=jnp.float32)
    p_ref[...] = _pack_gates(p)
    q_ref[...] = _pack_gates(q)


def _tc_tables1(x, wp, bp, wq):
    grid = N // BN
    return pl.pallas_call(
        _tables1_body,
        grid=(grid,),
        in_specs=[
            pl.BlockSpec((BN, C), lambda i: (i, 0)),
            pl.BlockSpec((C, C2), lambda i: (0, 0)),
            pl.BlockSpec((1, C2), lambda i: (0, 0)),
            pl.BlockSpec((C, C2), lambda i: (0, 0)),
        ],
        out_specs=[
            pl.BlockSpec((BN, C), lambda i: (i, 0)),
            pl.BlockSpec((BN, C), lambda i: (i, 0)),
        ],
        out_shape=[
            jax.ShapeDtypeStruct((N, C), jnp.int32),
            jax.ShapeDtypeStruct((N, C), jnp.int32),
        ],
    )(x, wp, bp, wq)


def _tables2_body(x_ref, a0_ref, a1_ref, a2_ref, a3_ref, wp_ref, bp_ref,
                  wq_ref, p_ref, q_ref, hsum_ref):
    i = pl.program_id(0)
    h = jnp.maximum(
        x_ref[...] + (a0_ref[...] + a1_ref[...]) + (a2_ref[...] + a3_ref[...]),
        0.0)
    p = jnp.dot(h, wp_ref[...], preferred_element_type=jnp.float32) + bp_ref[...]
    q = jnp.dot(h, wq_ref[...], preferred_element_type=jnp.float32)
    p_ref[...] = _pack_gates(p)
    q_ref[...] = _pack_gates(q)
    part = jnp.sum(h, axis=0, keepdims=True)

    @pl.when(i == 0)
    def _():
        hsum_ref[...] = part

    @pl.when(i != 0)
    def _():
        hsum_ref[...] += part


def _tc_tables2(x, a0, a1, a2, a3, wp, bp, wq):
    grid = N // BN
    return pl.pallas_call(
        _tables2_body,
        grid=(grid,),
        in_specs=[
            pl.BlockSpec((BN, C), lambda i: (i, 0)),
            pl.BlockSpec((BN, C), lambda i: (i, 0)),
            pl.BlockSpec((BN, C), lambda i: (i, 0)),
            pl.BlockSpec((BN, C), lambda i: (i, 0)),
            pl.BlockSpec((BN, C), lambda i: (i, 0)),
            pl.BlockSpec((C, C2), lambda i: (0, 0)),
            pl.BlockSpec((1, C2), lambda i: (0, 0)),
            pl.BlockSpec((C, C2), lambda i: (0, 0)),
        ],
        out_specs=[
            pl.BlockSpec((BN, C), lambda i: (i, 0)),
            pl.BlockSpec((BN, C), lambda i: (i, 0)),
            pl.BlockSpec((1, C), lambda i: (0, 0)),
        ],
        out_shape=[
            jax.ShapeDtypeStruct((N, C), jnp.int32),
            jax.ShapeDtypeStruct((N, C), jnp.int32),
            jax.ShapeDtypeStruct((1, C), jnp.float32),
        ],
    )(x, a0, a1, a2, a3, wp, bp, wq)


BE = 4000


def _softplus(s):
    return jnp.maximum(s, 0.0) + jnp.log1p(jnp.exp(-jnp.abs(s)))


def _sigmoid(f):
    return 0.5 * jnp.tanh(f * 0.5) + 0.5


def _edge_m_body(g_ref, ea_ref, we_ref, m_ref):
    gf, gs = _unpack_gates(g_ref[...])
    t = jnp.dot(ea_ref[...], we_ref[...], preferred_element_type=jnp.float32)
    f = gf + t[:, :C]
    s = gs + t[:, C:]
    m_ref[...] = _sigmoid(f) * _softplus(s)


def _tc_edge_m(g, ea, we, s):
    ne = g.shape[0]
    grid = ne // BE
    soff = s * (ES // BE)
    return pl.pallas_call(
        _edge_m_body,
        grid=(grid,),
        in_specs=[
            pl.BlockSpec((BE, C), lambda i: (i, 0)),
            pl.BlockSpec((BE, D), lambda i: (soff + i, 0)),
            pl.BlockSpec((D, C2), lambda i: (0, 0)),
        ],
        out_specs=pl.BlockSpec((BE, C), lambda i: (i, 0)),
        out_shape=jax.ShapeDtypeStruct((ne, C), jnp.float32),
    )(g, ea, we)


def _edge_msum_body(g_ref, ea_ref, we_ref, msum_ref):
    i = pl.program_id(0)
    gf, gs = _unpack_gates(g_ref[...])
    t = jnp.dot(ea_ref[...], we_ref[...], preferred_element_type=jnp.float32)
    f = gf + t[:, :C]
    s = gs + t[:, C:]
    part = jnp.sum(_sigmoid(f) * _softplus(s), axis=0, keepdims=True)

    @pl.when(i == 0)
    def _():
        msum_ref[...] = part

    @pl.when(i != 0)
    def _():
        msum_ref[...] += part


def _tc_edge_msum(g, ea, we, s):
    grid = g.shape[0] // BE
    soff = s * (ES // BE)
    return pl.pallas_call(
        _edge_msum_body,
        grid=(grid,),
        in_specs=[
            pl.BlockSpec((BE, C), lambda i: (i, 0)),
            pl.BlockSpec((BE, D), lambda i: (soff + i, 0)),
            pl.BlockSpec((D, C2), lambda i: (0, 0)),
        ],
        out_specs=pl.BlockSpec((1, C), lambda i: (0, 0)),
        out_shape=jax.ShapeDtypeStruct((1, C), jnp.float32),
    )(g, ea, we)


def _readout_body(sums_ref, wn_ref, nb_ref, o_ref):
    s = jnp.sum(sums_ref[...], axis=0, keepdims=True)
    o_ref[...] = jnp.dot(s, wn_ref[...], preferred_element_type=jnp.float32) + nb_ref[...]


def _tc_readout(sums, wnt, nb):
    nrows = sums.shape[0]
    return pl.pallas_call(
        _readout_body,
        in_specs=[
            pl.BlockSpec((nrows, C), lambda: (0, 0)),
            pl.BlockSpec((C, O), lambda: (0, 0)),
            pl.BlockSpec((1, O), lambda: (0, 0)),
        ],
        out_specs=pl.BlockSpec((1, O), lambda: (0, 0)),
        out_shape=jax.ShapeDtypeStruct((1, O), jnp.float32),
    )(sums, wnt, nb)


def _g_add_rows(abuf, bbuf):
    hi_mask = jnp.int32(-65536)
    bc = lax.bitcast_convert_type

    def add_row(e, _):
        for j in range(C // 16):
            a = abuf[e, pl.ds(j * 16, 16)]
            b = bbuf[e, pl.ds(j * 16, 16)]
            af = bc(a << 16, jnp.float32)
            as_ = bc(a & hi_mask, jnp.float32)
            bf_ = bc(b << 16, jnp.float32)
            bs_ = bc(b & hi_mask, jnp.float32)
            fu = bc(af + bf_, jnp.int32)
            su = bc(as_ + bs_, jnp.int32)
            abuf[e, pl.ds(j * 16, 16)] = (
                lax.shift_right_logical(fu, 16) | (su & hi_mask))
        return 0

    lax.fori_loop(0, CHUNK, add_row, 0)


def _make_sc_gather_body(slice_idx):
    def body(dst_hbm, src_hbm, p_hbm, q_hbm, g_hbm,
             didx, sidx, a0, a1, b0, b1,
             sem_a0, sem_a1, sem_b0, sem_b1, sem_s0, sem_s1):
        return _sc_gather_impl(
            slice_idx, dst_hbm, src_hbm, p_hbm, q_hbm, g_hbm,
            didx, sidx, a0, a1, b0, b1,
            sem_a0, sem_a1, sem_b0, sem_b1, sem_s0, sem_s1)
    return body


def _sc_gather_impl(slice_idx, dst_hbm, src_hbm, p_hbm, q_hbm, g_hbm,
                    didx, sidx, a0, a1, b0, b1,
                    sem_a0, sem_a1, sem_b0, sem_b1, sem_s0, sem_s1):
    wid = lax.axis_index("s") * 2 + lax.axis_index("c")
    pltpu.sync_copy(dst_hbm.at[slice_idx, wid], didx)
    pltpu.sync_copy(src_hbm.at[slice_idx, wid], sidx)

    bufs = ((a0, b0, sem_a0, sem_b0, sem_s0),
            (a1, b1, sem_a1, sem_b1, sem_s1))

    def issue_gather(t, p):
        a, b, sa, sb, _ = bufs[p]
        pltpu.async_copy(p_hbm.at[didx.at[t]], a, sa)
        pltpu.async_copy(q_hbm.at[sidx.at[t]], b, sb)

    def wait_gather(t, p):
        a, b, sa, sb, _ = bufs[p]
        pltpu.make_async_copy(p_hbm.at[didx.at[t]], a, sa).wait()
        pltpu.make_async_copy(q_hbm.at[sidx.at[t]], b, sb).wait()

    def wait_store(p):
        a, _, _, _, ss = bufs[p]
        pltpu.make_async_copy(a, g_hbm.at[pl.ds(0, CHUNK)], ss).wait()

    issue_gather(0, 0)

    def step(t, p, issue_next=True):
        q = 1 - p
        if issue_next:
            @pl.when(t >= 1)
            def _():
                wait_store(q)
            issue_gather(t + 1, q)

        wait_gather(t, p)
        a = bufs[p][0]
        _g_add_rows(a, bufs[p][1])
        pltpu.async_copy(a, g_hbm.at[pl.ds(wid * EWS + t * CHUNK, CHUNK)],
                         bufs[p][4])

    def pair(u, _):
        step(2 * u, 0)
        step(2 * u + 1, 1)
        return 0

    lax.fori_loop(0, NCHS // 2, pair, 0)
    step(NCHS - 1, 0, issue_next=False)
    wait_store(1)
    wait_store(0)


@functools.cache
def _sc_gather_kernel(slice_idx):
    return pl.kernel(
        _make_sc_gather_body(slice_idx),
        out_type=jax.ShapeDtypeStruct((ES, C), jnp.int32),
        mesh=_sc_mesh(),
        scratch_types=[
            pltpu.VMEM((NCHS, CHUNK), jnp.int32),
            pltpu.VMEM((NCHS, CHUNK), jnp.int32),
            pltpu.VMEM((CHUNK, C), jnp.int32),
            pltpu.VMEM((CHUNK, C), jnp.int32),
            pltpu.VMEM((CHUNK, C), jnp.int32),
            pltpu.VMEM((CHUNK, C), jnp.int32),
            pltpu.SemaphoreType.DMA,
            pltpu.SemaphoreType.DMA,
            pltpu.SemaphoreType.DMA,
            pltpu.SemaphoreType.DMA,
            pltpu.SemaphoreType.DMA,
            pltpu.SemaphoreType.DMA,
        ],
    )


def _sc_gather(s, dst4, src4, p, q):
    return _sc_gather_kernel(s)(dst4, src4, p, q)


def _make_sc_scatter_body(slices):
    k = len(slices)

    def body(dst_hbm, *rest):
        ms = rest[:k]
        out_hbm = rest[k]
        didx, zbuf, mb0, mb1, agg, sem_m0, sem_m1 = rest[k + 1:]
        cid = lax.axis_index("c")
        sid = lax.axis_index("s")
        wid = sid * 2 + cid

        def zrow(r, _):
            for j in range(C // 16):
                zbuf[r, pl.ds(j * 16, 16)] = jnp.zeros((16,), jnp.float32)
            return 0

        lax.fori_loop(0, ZROWS, zrow, 0)
        for rep in range(ROWS_PER_SUB // ZROWS):
            pltpu.sync_copy(
                zbuf, agg.at[pl.ds(sid * ROWS_PER_SUB + rep * ZROWS, ZROWS)])
        plsc.subcore_barrier()

        bufs = ((mb0, sem_m0), (mb1, sem_m1))

        for s, m_hbm in zip(slices, ms):
            pltpu.sync_copy(dst_hbm.at[s, wid], didx)
            def issue_load(t, p, m_hbm=m_hbm):
                b, sm = bufs[p]
                pltpu.async_copy(
                    m_hbm.at[pl.ds(wid * EWS + t * CHUNK, CHUNK)], b, sm)

            def wait_load(p, m_hbm=m_hbm):
                b, sm = bufs[p]
                pltpu.make_async_copy(m_hbm.at[pl.ds(0, CHUNK)], b, sm).wait()

            def step(t, p, issue_load=issue_load, wait_load=wait_load,
                     issue_next=True):
                if issue_next:
                    issue_load(t + 1, 1 - p)
                wait_load(p)
                pltpu.sync_copy(bufs[p][0], agg.at[didx.at[t]], add=True)

            issue_load(0, 0)

            def pair(u, _, step=step):
                step(2 * u, 0)
                step(2 * u + 1, 1)
                return 0

            lax.fori_loop(0, NCHS // 2, pair, 0)
            step(NCHS - 1, 0, issue_next=False)
        plsc.subcore_barrier()

        pltpu.sync_copy(
            agg.at[pl.ds(sid * ROWS_PER_SUB, ROWS_PER_SUB)],
            out_hbm.at[cid, pl.ds(sid * ROWS_PER_SUB, ROWS_PER_SUB)])

    return body


@functools.cache
def _sc_scatter_kernel(slices):
    return pl.kernel(
        _make_sc_scatter_body(slices),
        out_type=jax.ShapeDtypeStruct((2, NPAD, C), jnp.float32),
        mesh=_sc_mesh(),
        scratch_types=[
            pltpu.VMEM((NCHS, CHUNK), jnp.int32),
            pltpu.VMEM((ZROWS, C), jnp.float32),
            pltpu.VMEM((CHUNK, C), jnp.float32),
            pltpu.VMEM((CHUNK, C), jnp.float32),
            pltpu.VMEM_SHARED((NPAD, C), jnp.float32),
            pltpu.SemaphoreType.DMA,
            pltpu.SemaphoreType.DMA,
        ],
    )


def _sc_scatter(slices, dst4, ms):
    return _sc_scatter_kernel(tuple(slices))(dst4, *ms)


def kernel(x, edge_index, edge_attr, Wf1, bf1, Ws1, bs1, Wf2, bf2, Ws2, bs2, Wn, node_bias):
    dst = edge_index[1].astype(jnp.int32)
    src = edge_index[0].astype(jnp.int32)
    dst4 = dst.reshape(S, NW, NCHS, CHUNK)
    src4 = src.reshape(S, NW, NCHS, CHUNK)

    def split_w(Wf, Ws):
        wp = jnp.concatenate([Wf[:, :C].T, Ws[:, :C].T], axis=1)
        wq = jnp.concatenate([Wf[:, C:C2].T, Ws[:, C:C2].T], axis=1)
        we = jnp.concatenate([Wf[:, C2:].T, Ws[:, C2:].T], axis=1)
        return wp, wq, we

    wp1, wq1, we1 = split_w(Wf1, Ws1)
    wp2, wq2, we2 = split_w(Wf2, Ws2)
    bp1 = jnp.concatenate([bf1, bs1]).reshape(1, C2)
    bp2 = jnp.concatenate([bf2, bs2]).reshape(1, C2)

    p1, q1 = _tc_tables1(x, wp1, bp1, wq1)
    ms = []
    for s in range(S):
        g1 = _sc_gather(s, dst4, src4, p1, q1)
        ms.append(_tc_edge_m(g1, edge_attr, we1, s))
    parts_a = _sc_scatter(range(3), dst4, ms[:3])
    parts_b = _sc_scatter(range(3, S), dst4, ms[3:])

    p2, q2, hsum = _tc_tables2(x, parts_a[0], parts_a[1],
                               parts_b[0], parts_b[1], wp2, bp2, wq2)
    sums = [hsum]
    for s in range(S):
        g2 = _sc_gather(s, dst4, src4, p2, q2)
        sums.append(_tc_edge_msum(g2, edge_attr, we2, s))

    return _tc_readout(jnp.concatenate(sums, axis=0), Wn.T,
                       node_bias.reshape(1, O))


# speedup vs baseline: 1.2002x; 1.0042x over previous
import functools

import jax
import jax.numpy as jnp
from jax import lax
from jax.experimental import pallas as pl
from jax.experimental.pallas import tpu as pltpu
from jax.experimental.pallas import tpu_sc as plsc

N, E, C, D, O = 10000, 320000, 128, 16, 64
C2 = 2 * C

NW = 32
EW = E // NW
CHUNK = 80
NCH = EW // CHUNK
S = 5
ES = E // S
EWS = ES // NW
NCHS = EWS // CHUNK
NPAD = 10240
ROWS_PER_SUB = NPAD // 16
ZROWS = 128

@functools.cache
def _sc_mesh():
    return plsc.VectorSubcoreMesh(core_axis_name="c", subcore_axis_name="s",
                                  num_cores=2, num_subcores=16)


BN = 1000


def _pack_gates(t):
    lo = jax.lax.bitcast_convert_type(
        t[:, :C].astype(jnp.bfloat16), jnp.uint16).astype(jnp.uint32)
    hi = jax.lax.bitcast_convert_type(
        t[:, C:].astype(jnp.bfloat16), jnp.uint16).astype(jnp.uint32)
    return jax.lax.bitcast_convert_type(lo | (hi << 16), jnp.int32)


def _unpack_gates(u):
    ub = jax.lax.bitcast_convert_type(u, jnp.uint32)
    f = jax.lax.bitcast_convert_type(ub << 16, jnp.float32)
    s = jax.lax.bitcast_convert_type(ub & jnp.uint32(0xFFFF0000), jnp.float32)
    return f, s


def _tables1_body(x_ref, wp_ref, bp_ref, wq_ref, p_ref, q_ref):
    x = x_ref[...]
    p = jnp.dot(x, wp_ref[...], preferred_element_type=jnp.float32) + bp_ref[...]
    q = jnp.dot(x, wq_ref[...], preferred_element_type=jnp.float32)
    p_ref[...] = _pack_gates(p)
    q_ref[...] = _pack_gates(q)


def _tc_tables1(x, wp, bp, wq):
    grid = N // BN
    return pl.pallas_call(
        _tables1_body,
        grid=(grid,),
        in_specs=[
            pl.BlockSpec((BN, C), lambda i: (i, 0)),
            pl.BlockSpec((C, C2), lambda i: (0, 0)),
            pl.BlockSpec((1, C2), lambda i: (0, 0)),
            pl.BlockSpec((C, C2), lambda i: (0, 0)),
        ],
        out_specs=[
            pl.BlockSpec((BN, C), lambda i: (i, 0)),
            pl.BlockSpec((BN, C), lambda i: (i, 0)),
        ],
        out_shape=[
            jax.ShapeDtypeStruct((N, C), jnp.int32),
            jax.ShapeDtypeStruct((N, C), jnp.int32),
        ],
    )(x, wp, bp, wq)


def _tables2_body(x_ref, a0_ref, a1_ref, a2_ref, a3_ref, wp_ref, bp_ref,
                  wq_ref, p_ref, q_ref, hsum_ref):
    i = pl.program_id(0)
    h = jnp.maximum(
        x_ref[...] + (a0_ref[...] + a1_ref[...]) + (a2_ref[...] + a3_ref[...]),
        0.0)
    p = jnp.dot(h, wp_ref[...], preferred_element_type=jnp.float32) + bp_ref[...]
    q = jnp.dot(h, wq_ref[...], preferred_element_type=jnp.float32)
    p_ref[...] = _pack_gates(p)
    q_ref[...] = _pack_gates(q)
    part = jnp.sum(h, axis=0, keepdims=True)

    @pl.when(i == 0)
    def _():
        hsum_ref[...] = part

    @pl.when(i != 0)
    def _():
        hsum_ref[...] += part


def _tc_tables2(x, a0, a1, a2, a3, wp, bp, wq):
    grid = N // BN
    return pl.pallas_call(
        _tables2_body,
        grid=(grid,),
        in_specs=[
            pl.BlockSpec((BN, C), lambda i: (i, 0)),
            pl.BlockSpec((BN, C), lambda i: (i, 0)),
            pl.BlockSpec((BN, C), lambda i: (i, 0)),
            pl.BlockSpec((BN, C), lambda i: (i, 0)),
            pl.BlockSpec((BN, C), lambda i: (i, 0)),
            pl.BlockSpec((C, C2), lambda i: (0, 0)),
            pl.BlockSpec((1, C2), lambda i: (0, 0)),
            pl.BlockSpec((C, C2), lambda i: (0, 0)),
        ],
        out_specs=[
            pl.BlockSpec((BN, C), lambda i: (i, 0)),
            pl.BlockSpec((BN, C), lambda i: (i, 0)),
            pl.BlockSpec((1, C), lambda i: (0, 0)),
        ],
        out_shape=[
            jax.ShapeDtypeStruct((N, C), jnp.int32),
            jax.ShapeDtypeStruct((N, C), jnp.int32),
            jax.ShapeDtypeStruct((1, C), jnp.float32),
        ],
    )(x, a0, a1, a2, a3, wp, bp, wq)


BE = 8000


def _softplus(s):
    return jnp.maximum(s, 0.0) + jnp.log1p(jnp.exp(-jnp.abs(s)))


def _sigmoid(f):
    return 0.5 * jnp.tanh(f * 0.5) + 0.5


def _edge_m_body(g_ref, ea_ref, we_ref, m_ref):
    gf, gs = _unpack_gates(g_ref[...])
    t = jnp.dot(ea_ref[...], we_ref[...], preferred_element_type=jnp.float32)
    f = gf + t[:, :C]
    s = gs + t[:, C:]
    m_ref[...] = _sigmoid(f) * _softplus(s)


def _tc_edge_m(g, ea, we, s):
    ne = g.shape[0]
    grid = ne // BE
    soff = s * (ES // BE)
    return pl.pallas_call(
        _edge_m_body,
        grid=(grid,),
        in_specs=[
            pl.BlockSpec((BE, C), lambda i: (i, 0)),
            pl.BlockSpec((BE, D), lambda i: (soff + i, 0)),
            pl.BlockSpec((D, C2), lambda i: (0, 0)),
        ],
        out_specs=pl.BlockSpec((BE, C), lambda i: (i, 0)),
        out_shape=jax.ShapeDtypeStruct((ne, C), jnp.float32),
    )(g, ea, we)


def _edge_msum_body(g_ref, ea_ref, we_ref, msum_ref):
    i = pl.program_id(0)
    gf, gs = _unpack_gates(g_ref[...])
    t = jnp.dot(ea_ref[...], we_ref[...], preferred_element_type=jnp.float32)
    f = gf + t[:, :C]
    s = gs + t[:, C:]
    part = jnp.sum(_sigmoid(f) * _softplus(s), axis=0, keepdims=True)

    @pl.when(i == 0)
    def _():
        msum_ref[...] = part

    @pl.when(i != 0)
    def _():
        msum_ref[...] += part


def _tc_edge_msum(g, ea, we, s):
    grid = g.shape[0] // BE
    soff = s * (ES // BE)
    return pl.pallas_call(
        _edge_msum_body,
        grid=(grid,),
        in_specs=[
            pl.BlockSpec((BE, C), lambda i: (i, 0)),
            pl.BlockSpec((BE, D), lambda i: (soff + i, 0)),
            pl.BlockSpec((D, C2), lambda i: (0, 0)),
        ],
        out_specs=pl.BlockSpec((1, C), lambda i: (0, 0)),
        out_shape=jax.ShapeDtypeStruct((1, C), jnp.float32),
    )(g, ea, we)


def _readout_body(sums_ref, wn_ref, nb_ref, o_ref):
    s = jnp.sum(sums_ref[...], axis=0, keepdims=True)
    o_ref[...] = jnp.dot(s, wn_ref[...], preferred_element_type=jnp.float32) + nb_ref[...]


def _tc_readout(sums, wnt, nb):
    nrows = sums.shape[0]
    return pl.pallas_call(
        _readout_body,
        in_specs=[
            pl.BlockSpec((nrows, C), lambda: (0, 0)),
            pl.BlockSpec((C, O), lambda: (0, 0)),
            pl.BlockSpec((1, O), lambda: (0, 0)),
        ],
        out_specs=pl.BlockSpec((1, O), lambda: (0, 0)),
        out_shape=jax.ShapeDtypeStruct((1, O), jnp.float32),
    )(sums, wnt, nb)


def _g_add_rows(abuf, bbuf):
    hi_mask = jnp.int32(-65536)
    bc = lax.bitcast_convert_type

    def add_row(e, _):
        for j in range(C // 16):
            a = abuf[e, pl.ds(j * 16, 16)]
            b = bbuf[e, pl.ds(j * 16, 16)]
            af = bc(a << 16, jnp.float32)
            as_ = bc(a & hi_mask, jnp.float32)
            bf_ = bc(b << 16, jnp.float32)
            bs_ = bc(b & hi_mask, jnp.float32)
            fu = bc(af + bf_, jnp.int32)
            su = bc(as_ + bs_, jnp.int32)
            abuf[e, pl.ds(j * 16, 16)] = (
                lax.shift_right_logical(fu, 16) | (su & hi_mask))
        return 0

    lax.fori_loop(0, CHUNK, add_row, 0)


def _make_sc_gather_body(slice_idx):
    def body(dst_hbm, src_hbm, p_hbm, q_hbm, g_hbm,
             didx, sidx, a0, a1, b0, b1,
             sem_a0, sem_a1, sem_b0, sem_b1, sem_s0, sem_s1):
        return _sc_gather_impl(
            slice_idx, dst_hbm, src_hbm, p_hbm, q_hbm, g_hbm,
            didx, sidx, a0, a1, b0, b1,
            sem_a0, sem_a1, sem_b0, sem_b1, sem_s0, sem_s1)
    return body


def _sc_gather_impl(slice_idx, dst_hbm, src_hbm, p_hbm, q_hbm, g_hbm,
                    didx, sidx, a0, a1, b0, b1,
                    sem_a0, sem_a1, sem_b0, sem_b1, sem_s0, sem_s1):
    wid = lax.axis_index("s") * 2 + lax.axis_index("c")
    pltpu.sync_copy(dst_hbm.at[slice_idx, wid], didx)
    pltpu.sync_copy(src_hbm.at[slice_idx, wid], sidx)

    bufs = ((a0, b0, sem_a0, sem_b0, sem_s0),
            (a1, b1, sem_a1, sem_b1, sem_s1))

    def issue_gather(t, p):
        a, b, sa, sb, _ = bufs[p]
        pltpu.async_copy(p_hbm.at[didx.at[t]], a, sa)
        pltpu.async_copy(q_hbm.at[sidx.at[t]], b, sb)

    def wait_gather(t, p):
        a, b, sa, sb, _ = bufs[p]
        pltpu.make_async_copy(p_hbm.at[didx.at[t]], a, sa).wait()
        pltpu.make_async_copy(q_hbm.at[sidx.at[t]], b, sb).wait()

    def wait_store(p):
        a, _, _, _, ss = bufs[p]
        pltpu.make_async_copy(a, g_hbm.at[pl.ds(0, CHUNK)], ss).wait()

    issue_gather(0, 0)

    def step(t, p, issue_next=True):
        q = 1 - p
        if issue_next:
            @pl.when(t >= 1)
            def _():
                wait_store(q)
            issue_gather(t + 1, q)

        wait_gather(t, p)
        a = bufs[p][0]
        _g_add_rows(a, bufs[p][1])
        pltpu.async_copy(a, g_hbm.at[pl.ds(wid * EWS + t * CHUNK, CHUNK)],
                         bufs[p][4])

    def pair(u, _):
        step(2 * u, 0)
        step(2 * u + 1, 1)
        return 0

    lax.fori_loop(0, NCHS // 2, pair, 0)
    step(NCHS - 1, 0, issue_next=False)
    wait_store(1)
    wait_store(0)


@functools.cache
def _sc_gather_kernel(slice_idx):
    return pl.kernel(
        _make_sc_gather_body(slice_idx),
        out_type=jax.ShapeDtypeStruct((ES, C), jnp.int32),
        mesh=_sc_mesh(),
        scratch_types=[
            pltpu.VMEM((NCHS, CHUNK), jnp.int32),
            pltpu.VMEM((NCHS, CHUNK), jnp.int32),
            pltpu.VMEM((CHUNK, C), jnp.int32),
            pltpu.VMEM((CHUNK, C), jnp.int32),
            pltpu.VMEM((CHUNK, C), jnp.int32),
            pltpu.VMEM((CHUNK, C), jnp.int32),
            pltpu.SemaphoreType.DMA,
            pltpu.SemaphoreType.DMA,
            pltpu.SemaphoreType.DMA,
            pltpu.SemaphoreType.DMA,
            pltpu.SemaphoreType.DMA,
            pltpu.SemaphoreType.DMA,
        ],
    )


def _sc_gather(s, dst4, src4, p, q):
    return _sc_gather_kernel(s)(dst4, src4, p, q)


def _make_sc_scatter_body(slices):
    k = len(slices)

    def body(dst_hbm, *rest):
        ms = rest[:k]
        out_hbm = rest[k]
        didx, zbuf, mb0, mb1, agg, sem_m0, sem_m1 = rest[k + 1:]
        cid = lax.axis_index("c")
        sid = lax.axis_index("s")
        wid = sid * 2 + cid

        def zrow(r, _):
            for j in range(C // 16):
                zbuf[r, pl.ds(j * 16, 16)] = jnp.zeros((16,), jnp.float32)
            return 0

        lax.fori_loop(0, ZROWS, zrow, 0)
        for rep in range(ROWS_PER_SUB // ZROWS):
            pltpu.sync_copy(
                zbuf, agg.at[pl.ds(sid * ROWS_PER_SUB + rep * ZROWS, ZROWS)])
        plsc.subcore_barrier()

        bufs = ((mb0, sem_m0), (mb1, sem_m1))

        for s, m_hbm in zip(slices, ms):
            pltpu.sync_copy(dst_hbm.at[s, wid], didx)
            def issue_load(t, p, m_hbm=m_hbm):
                b, sm = bufs[p]
                pltpu.async_copy(
                    m_hbm.at[pl.ds(wid * EWS + t * CHUNK, CHUNK)], b, sm)

            def wait_load(p, m_hbm=m_hbm):
                b, sm = bufs[p]
                pltpu.make_async_copy(m_hbm.at[pl.ds(0, CHUNK)], b, sm).wait()

            def step(t, p, issue_load=issue_load, wait_load=wait_load,
                     issue_next=True):
                if issue_next:
                    issue_load(t + 1, 1 - p)
                wait_load(p)
                pltpu.sync_copy(bufs[p][0], agg.at[didx.at[t]], add=True)

            issue_load(0, 0)

            def pair(u, _, step=step):
                step(2 * u, 0)
                step(2 * u + 1, 1)
                return 0

            lax.fori_loop(0, NCHS // 2, pair, 0)
            step(NCHS - 1, 0, issue_next=False)
        plsc.subcore_barrier()

        pltpu.sync_copy(
            agg.at[pl.ds(sid * ROWS_PER_SUB, ROWS_PER_SUB)],
            out_hbm.at[cid, pl.ds(sid * ROWS_PER_SUB, ROWS_PER_SUB)])

    return body


@functools.cache
def _sc_scatter_kernel(slices):
    return pl.kernel(
        _make_sc_scatter_body(slices),
        out_type=jax.ShapeDtypeStruct((2, NPAD, C), jnp.float32),
        mesh=_sc_mesh(),
        scratch_types=[
            pltpu.VMEM((NCHS, CHUNK), jnp.int32),
            pltpu.VMEM((ZROWS, C), jnp.float32),
            pltpu.VMEM((CHUNK, C), jnp.float32),
            pltpu.VMEM((CHUNK, C), jnp.float32),
            pltpu.VMEM_SHARED((NPAD, C), jnp.float32),
            pltpu.SemaphoreType.DMA,
            pltpu.SemaphoreType.DMA,
        ],
    )


def _sc_scatter(slices, dst4, ms):
    return _sc_scatter_kernel(tuple(slices))(dst4, *ms)


def kernel(x, edge_index, edge_attr, Wf1, bf1, Ws1, bs1, Wf2, bf2, Ws2, bs2, Wn, node_bias):
    dst = edge_index[1].astype(jnp.int32)
    src = edge_index[0].astype(jnp.int32)
    dst4 = dst.reshape(S, NW, NCHS, CHUNK)
    src4 = src.reshape(S, NW, NCHS, CHUNK)

    def split_w(Wf, Ws):
        wp = jnp.concatenate([Wf[:, :C].T, Ws[:, :C].T], axis=1)
        wq = jnp.concatenate([Wf[:, C:C2].T, Ws[:, C:C2].T], axis=1)
        we = jnp.concatenate([Wf[:, C2:].T, Ws[:, C2:].T], axis=1)
        return wp, wq, we

    wp1, wq1, we1 = split_w(Wf1, Ws1)
    wp2, wq2, we2 = split_w(Wf2, Ws2)
    bp1 = jnp.concatenate([bf1, bs1]).reshape(1, C2)
    bp2 = jnp.concatenate([bf2, bs2]).reshape(1, C2)

    p1, q1 = _tc_tables1(x, wp1, bp1, wq1)
    ms = []
    for s in range(S):
        g1 = _sc_gather(s, dst4, src4, p1, q1)
        ms.append(_tc_edge_m(g1, edge_attr, we1, s))
    parts_a = _sc_scatter(range(3), dst4, ms[:3])
    parts_b = _sc_scatter(range(3, S), dst4, ms[3:])

    p2, q2, hsum = _tc_tables2(x, parts_a[0], parts_a[1],
                               parts_b[0], parts_b[1], wp2, bp2, wq2)
    sums = [hsum]
    for s in range(S):
        g2 = _sc_gather(s, dst4, src4, p2, q2)
        sums.append(_tc_edge_msum(g2, edge_attr, we2, s))

    return _tc_readout(jnp.concatenate(sums, axis=0), Wn.T,
                       node_bias.reshape(1, O))
